# Initial kernel scaffold; baseline (speedup 1.0000x reference)
import jax
import jax.numpy as jnp
from jax.experimental import pallas as pl


def kernel(point_coords, point_features, rois, wg0a, wg0b, wg1a, wg1b, wfc1, wfc2, wcls1, wcls2, wcls3, bcls3, wiou1, wiou2, wiou3, biou3, wreg1, wreg2, wreg3, breg3):
    raise NotImplementedError("write your pallas kernel here")


# speedup vs baseline: 7.7885x; 7.7885x over previous
import functools

import jax
import jax.numpy as jnp
from jax import lax
from jax.experimental import pallas as pl
from jax.experimental.pallas import tpu as pltpu
from jax.experimental.pallas import tpu_sc as plsc

NQ = 8192
NP = 8192
QBLK = 256
NBLK = NQ // QBLK
PBLK = 1024
K1 = 32
K0 = 16
KT = K1 + K0
R0SQ = 0.8 * 0.8
R1SQ = 1.6 * 1.6

NW = 32
TOTAL = K1 * NQ
PERW = TOTAL // NW
CH = 512
NCH = PERW // CH


def _prep_kernel(x_ref, nx_ref, wg0a_ref, wg1a_ref, pb_ref, tab_ref, qa_ref):
    x = x_ref[...]
    xx = x[:, 0:8]
    row8 = lax.broadcasted_iota(jnp.int32, (8, 8), 0)
    col8 = lax.broadcasted_iota(jnp.int32, (8, 8), 1)
    s1 = jnp.where((row8 == col8) & (col8 < 3), 1.0, 0.0)
    s2 = jnp.where((col8 == 4) & (row8 < 3), 1.0, 0.0)
    cb = jnp.where((row8 == 0) & (col8 == 3), 1.0, 0.0)[0:1, :]
    dot = functools.partial(lax.dot, precision=lax.Precision.HIGHEST,
                            preferred_element_type=jnp.float32)
    pb_ref[...] = dot(xx, s1) + dot(xx * xx, s2) + cb
    dotd = functools.partial(lax.dot, preferred_element_type=jnp.float32)
    tab_ref[...] = jnp.concatenate(
        [dotd(x, wg1a_ref[...]), dotd(x, wg0a_ref[...])], axis=1)
    n3 = nx_ref[...]
    row3 = lax.broadcasted_iota(jnp.int32, (3, 8), 0)
    col3 = lax.broadcasted_iota(jnp.int32, (3, 8), 1)
    q1 = jnp.where(row3 == col3, 2.0, 0.0)
    q2 = jnp.where(col3 == 3, -1.0, 0.0)
    qc = jnp.where((row8 == 0) & (col8 == 4), -1.0, 0.0)[0:1, :]
    qa_ref[...] = dot(n3, q1) + dot(n3 * n3, q2) + qc


def _select_kernel(qa_ref, pb_ref, idx_ref, vm_ref):
    qa = qa_ref[...]
    pb = pb_ref[...]
    sneg = lax.dot_general(qa, pb, (((1,), (1,)), ((), ())),
                           precision=lax.Precision.HIGHEST,
                           preferred_element_type=jnp.float32)
    iota = lax.broadcasted_iota(jnp.int32, (QBLK, NP), 1)
    idxs = []
    d2s = []
    for _ in range(K1):
        mk = jnp.max(sneg, axis=1, keepdims=True)
        ik = jnp.argmax(sneg, axis=1).astype(jnp.int32)[:, None]
        idxs.append(ik)
        d2s.append(-mk)
        sneg = jnp.where(iota == ik, -jnp.inf, sneg)
    idx_ref[...] = jnp.concatenate(idxs, axis=1)
    v1 = [jnp.where(d <= R1SQ, 1.0, 0.0) for d in d2s]
    v0 = [jnp.where(d <= R0SQ, 1.0, 0.0) for d in d2s[:K0]]
    vm_ref[...] = jnp.concatenate(v1 + v0, axis=1)


def _gather_body(tab_ref, idx_ref, out_ref, idx_v, rows_v, sem):
    wid = lax.axis_index("s") * 2 + lax.axis_index("c")
    base = wid * PERW
    for j in range(NCH):
        off = base + j * CH
        pltpu.sync_copy(idx_ref.at[pl.ds(off, CH)], idx_v)
        pltpu.async_copy(tab_ref.at[idx_v], rows_v, sem).wait()
        pltpu.sync_copy(rows_v, out_ref.at[pl.ds(off, CH)])


def _sc_gather(tab, idx_flat):
    call = pl.kernel(
        _gather_body,
        out_type=jax.ShapeDtypeStruct((TOTAL, 128), jnp.float32),
        mesh=plsc.VectorSubcoreMesh(core_axis_name="c", subcore_axis_name="s",
                                    num_cores=2, num_subcores=16),
        scratch_types=[
            pltpu.VMEM((CH,), jnp.int32),
            pltpu.VMEM((CH, 128), jnp.float32),
            pltpu.SemaphoreType.DMA,
        ],
    )
    return call(tab, idx_flat)


def _mlp_kernel(qa_ref, rows_ref, vm_ref, wg0a_ref, wg1a_ref,
                w0b_ref, w1b_ref, out_ref):
    dot = functools.partial(lax.dot, preferred_element_type=jnp.float32)
    qa = qa_ref[...]
    q3 = qa[:, 0:3] * 0.5
    b0 = dot(q3, wg0a_ref[0:3, :])
    b1 = dot(q3, wg1a_ref[0:3, :])
    w0b = w0b_ref[...]
    w1b = w1b_ref[...]
    vm = vm_ref[...]
    acc0 = jnp.zeros((QBLK, 64), jnp.float32)
    acc1 = jnp.zeros((QBLK, 64), jnp.float32)
    for k in range(K1):
        row = rows_ref[:, k, :]
        h = jax.nn.relu(dot(jax.nn.relu(row[:, 0:64] - b1), w1b))
        acc1 = jnp.maximum(acc1, jnp.where(vm[:, k:k + 1] > 0, h, 0.0))
        if k < K0:
            h = jax.nn.relu(dot(jax.nn.relu(row[:, 64:128] - b0), w0b))
            acc0 = jnp.maximum(
                acc0, jnp.where(vm[:, K1 + k:K1 + k + 1] > 0, h, 0.0))
    out_ref[...] = jnp.concatenate([acc0, acc1], axis=1)


def _heads_kernel(x_ref, wfc1_ref, wfc2_ref, wcls1_ref, wcls2_ref, wcls3_ref,
                  bcls3_ref, wiou1_ref, wiou2_ref, wiou3_ref, biou3_ref,
                  wreg1_ref, wreg2_ref, wreg3_ref, breg3_ref, out_ref):
    dot = functools.partial(lax.dot, preferred_element_type=jnp.float32)
    x = x_ref[...]
    h = jax.nn.relu(dot(x, wfc1_ref[...]))
    h = jax.nn.relu(dot(h, wfc2_ref[...]))
    cls = dot(jax.nn.relu(dot(jax.nn.relu(dot(h, wcls1_ref[...])),
                              wcls2_ref[...])), wcls3_ref[...]) + bcls3_ref[...]
    iou = dot(jax.nn.relu(dot(jax.nn.relu(dot(h, wiou1_ref[...])),
                              wiou2_ref[...])), wiou3_ref[...]) + biou3_ref[...]
    reg = dot(jax.nn.relu(dot(jax.nn.relu(dot(h, wreg1_ref[...])),
                              wreg2_ref[...])), wreg3_ref[...]) + breg3_ref[...]
    out_ref[...] = jnp.concatenate([cls, iou, reg], axis=1)


def kernel(point_coords, point_features, rois, wg0a, wg0b, wg1a, wg1b,
           wfc1, wfc2, wcls1, wcls2, wcls3, bcls3, wiou1, wiou2, wiou3,
           biou3, wreg1, wreg2, wreg3, breg3):
    f32 = jnp.float32
    x = jnp.concatenate([point_coords, point_features], axis=1)

    gidx = jnp.stack(jnp.meshgrid(jnp.arange(4), jnp.arange(4), jnp.arange(4),
                                  indexing='ij'), axis=-1).reshape(-1, 3)
    gidx = gidx.astype(f32)
    size = rois[:, 3:6]
    local = (gidx[None] + 0.5) / 4 * size[:, None] - size[:, None] / 2
    ang = rois[:, 6]
    c = jnp.cos(ang)
    s = jnp.sin(ang)
    z = jnp.zeros_like(c)
    o = jnp.ones_like(c)
    rot = jnp.stack([c, s, z, -s, c, z, z, z, o], axis=1).reshape(-1, 3, 3)
    gp = jnp.einsum('bpc,bcd->bpd', local, rot) + rois[:, None, 0:3]
    nxyz = gp.reshape(NQ, 3)

    pb, tab, qa = pl.pallas_call(
        _prep_kernel,
        grid=(NP // PBLK,),
        in_specs=[
            pl.BlockSpec((PBLK, 131), lambda i: (i, 0)),
            pl.BlockSpec((PBLK, 3), lambda i: (i, 0)),
            pl.BlockSpec((131, 64), lambda i: (0, 0)),
            pl.BlockSpec((131, 64), lambda i: (0, 0)),
        ],
        out_specs=[
            pl.BlockSpec((PBLK, 8), lambda i: (i, 0)),
            pl.BlockSpec((PBLK, 128), lambda i: (i, 0)),
            pl.BlockSpec((PBLK, 8), lambda i: (i, 0)),
        ],
        out_shape=[
            jax.ShapeDtypeStruct((NP, 8), f32),
            jax.ShapeDtypeStruct((NP, 128), f32),
            jax.ShapeDtypeStruct((NQ, 8), f32),
        ],
    )(x, nxyz, wg0a, wg1a)

    idx48, vmask = pl.pallas_call(
        _select_kernel,
        grid=(NBLK,),
        in_specs=[
            pl.BlockSpec((QBLK, 8), lambda i: (i, 0)),
            pl.BlockSpec((NP, 8), lambda i: (0, 0)),
        ],
        out_specs=[
            pl.BlockSpec((QBLK, K1), lambda i: (i, 0)),
            pl.BlockSpec((QBLK, KT), lambda i: (i, 0)),
        ],
        out_shape=[
            jax.ShapeDtypeStruct((NQ, K1), jnp.int32),
            jax.ShapeDtypeStruct((NQ, KT), f32),
        ],
    )(qa, pb)

    idx_flat = idx48.reshape(TOTAL)
    rows = _sc_gather(tab, idx_flat)
    rows3 = rows.reshape(NQ, K1, 128)

    pooled = pl.pallas_call(
        _mlp_kernel,
        grid=(NBLK,),
        in_specs=[
            pl.BlockSpec((QBLK, 8), lambda i: (i, 0)),
            pl.BlockSpec((QBLK, K1, 128), lambda i: (i, 0, 0)),
            pl.BlockSpec((QBLK, KT), lambda i: (i, 0)),
            pl.BlockSpec((131, 64), lambda i: (0, 0)),
            pl.BlockSpec((131, 64), lambda i: (0, 0)),
            pl.BlockSpec((64, 64), lambda i: (0, 0)),
            pl.BlockSpec((64, 64), lambda i: (0, 0)),
        ],
        out_specs=pl.BlockSpec((QBLK, 128), lambda i: (i, 0)),
        out_shape=jax.ShapeDtypeStruct((NQ, 128), f32),
    )(qa, rows3, vmask, wg0a, wg1a, wg0b, wg1b)

    xfc = pooled.reshape(128, 64 * 128)
    out = pl.pallas_call(
        _heads_kernel,
        out_shape=jax.ShapeDtypeStruct((128, 9), f32),
    )(xfc, wfc1, wfc2, wcls1, wcls2, wcls3, bcls3.reshape(1, 1),
      wiou1, wiou2, wiou3, biou3.reshape(1, 1),
      wreg1, wreg2, wreg3, breg3.reshape(1, 7))
    return out


# speedup vs baseline: 9.0040x; 1.1561x over previous
import functools

import jax
import jax.numpy as jnp
from jax import lax
from jax.experimental import pallas as pl
from jax.experimental.pallas import tpu as pltpu
from jax.experimental.pallas import tpu_sc as plsc

NQ = 8192
NP = 8192
QBLK = 512
NBLK = NQ // QBLK
PBLK = 1024
K1 = 32
K0 = 16
KT = K1 + K0
R0SQ = 0.8 * 0.8
R1SQ = 1.6 * 1.6

NW = 32
TOTAL = K1 * NQ
PERW = TOTAL // NW
CH = 512
NCH = PERW // CH


def _prep_kernel(x_ref, nx_ref, wg0a_ref, wg1a_ref, pb_ref, tab_ref, qa_ref):
    x = x_ref[...]
    xx = x[:, 0:8]
    row8 = lax.broadcasted_iota(jnp.int32, (8, 8), 0)
    col8 = lax.broadcasted_iota(jnp.int32, (8, 8), 1)
    s1 = jnp.where((row8 == col8) & (col8 < 3), 1.0, 0.0)
    s2 = jnp.where((col8 == 4) & (row8 < 3), 1.0, 0.0)
    cb = jnp.where((row8 == 0) & (col8 == 3), 1.0, 0.0)[0:1, :]
    dot = functools.partial(lax.dot, precision=lax.Precision.HIGHEST,
                            preferred_element_type=jnp.float32)
    pb_ref[...] = dot(xx, s1) + dot(xx * xx, s2) + cb
    dotd = functools.partial(lax.dot, preferred_element_type=jnp.float32)
    tab_ref[...] = jnp.concatenate(
        [dotd(x, wg1a_ref[...]), dotd(x, wg0a_ref[...])], axis=1)
    n3 = nx_ref[...]
    row3 = lax.broadcasted_iota(jnp.int32, (3, 8), 0)
    col3 = lax.broadcasted_iota(jnp.int32, (3, 8), 1)
    q1 = jnp.where(row3 == col3, 2.0, 0.0)
    q2 = jnp.where(col3 == 3, -1.0, 0.0)
    qc = jnp.where((row8 == 0) & (col8 == 4), -1.0, 0.0)[0:1, :]
    qa_ref[...] = dot(n3, q1) + dot(n3 * n3, q2) + qc


def _select_kernel(qa_ref, pb_ref, idx_ref, vm_ref):
    qa = qa_ref[...]
    pb = pb_ref[...]
    sneg = lax.dot_general(qa, pb, (((1,), (1,)), ((), ())),
                           precision=lax.Precision.HIGHEST,
                           preferred_element_type=jnp.float32)
    c1 = jnp.sum(jnp.where(sneg >= -R1SQ, 1, 0), axis=1,
                 keepdims=True).astype(jnp.int32)
    c0 = jnp.sum(jnp.where(sneg >= -R0SQ, 1, 0), axis=1,
                 keepdims=True).astype(jnp.int32)
    i48 = lax.broadcasted_iota(jnp.int32, (QBLK, KT), 1)
    vm1 = jnp.where(i48 < c1, 1.0, 0.0)
    vm0 = jnp.where(i48 - K1 < c0, 1.0, 0.0)
    vm_ref[...] = jnp.where(i48 < K1, vm1, vm0)

    iota = lax.broadcasted_iota(jnp.int32, (QBLK, NP), 1)
    idxs = []
    for _ in range(K1):
        ik = jnp.argmax(sneg, axis=1).astype(jnp.int32)[:, None]
        idxs.append(ik)
        sneg = jnp.where(iota == ik, -jnp.inf, sneg)
    idx_ref[...] = jnp.concatenate(idxs, axis=1)


def _gather_body(tab_ref, idx_ref, out_ref, idx_v, rows_v, sem):
    wid = lax.axis_index("s") * 2 + lax.axis_index("c")
    base = wid * PERW
    for j in range(NCH):
        off = base + j * CH
        pltpu.sync_copy(idx_ref.at[pl.ds(off, CH)], idx_v)
        pltpu.async_copy(tab_ref.at[idx_v], rows_v, sem).wait()
        pltpu.sync_copy(rows_v, out_ref.at[pl.ds(off, CH)])


def _sc_gather(tab, idx_flat):
    call = pl.kernel(
        _gather_body,
        out_type=jax.ShapeDtypeStruct((TOTAL, 128), jnp.float32),
        mesh=plsc.VectorSubcoreMesh(core_axis_name="c", subcore_axis_name="s",
                                    num_cores=2, num_subcores=16),
        scratch_types=[
            pltpu.VMEM((CH,), jnp.int32),
            pltpu.VMEM((CH, 128), jnp.float32),
            pltpu.SemaphoreType.DMA,
        ],
    )
    return call(tab, idx_flat)


def _mlp_kernel(qa_ref, rows_ref, vm_ref, wg0a_ref, wg1a_ref,
                w0b_ref, w1b_ref, out_ref):
    dot = functools.partial(lax.dot, preferred_element_type=jnp.float32)
    qa = qa_ref[...]
    q3 = qa[:, 0:3] * 0.5
    b0 = dot(q3, wg0a_ref[0:3, :])
    b1 = dot(q3, wg1a_ref[0:3, :])
    w0b = w0b_ref[...]
    w1b = w1b_ref[...]
    vm = vm_ref[...]
    acc0 = jnp.zeros((QBLK, 64), jnp.float32)
    acc1 = jnp.zeros((QBLK, 64), jnp.float32)
    for k in range(K1):
        row = rows_ref[:, k, :]
        h = jax.nn.relu(dot(jax.nn.relu(row[:, 0:64] - b1), w1b))
        acc1 = jnp.maximum(acc1, jnp.where(vm[:, k:k + 1] > 0, h, 0.0))
        if k < K0:
            h = jax.nn.relu(dot(jax.nn.relu(row[:, 64:128] - b0), w0b))
            acc0 = jnp.maximum(
                acc0, jnp.where(vm[:, K1 + k:K1 + k + 1] > 0, h, 0.0))
    out_ref[...] = jnp.concatenate([acc0, acc1], axis=1)


def _heads_kernel(x_ref, wfc1_ref, wfc2_ref, wcls1_ref, wcls2_ref, wcls3_ref,
                  bcls3_ref, wiou1_ref, wiou2_ref, wiou3_ref, biou3_ref,
                  wreg1_ref, wreg2_ref, wreg3_ref, breg3_ref, out_ref):
    dot = functools.partial(lax.dot, preferred_element_type=jnp.float32)
    x = x_ref[...]
    h = jax.nn.relu(dot(x, wfc1_ref[...]))
    h = jax.nn.relu(dot(h, wfc2_ref[...]))
    cls = dot(jax.nn.relu(dot(jax.nn.relu(dot(h, wcls1_ref[...])),
                              wcls2_ref[...])), wcls3_ref[...]) + bcls3_ref[...]
    iou = dot(jax.nn.relu(dot(jax.nn.relu(dot(h, wiou1_ref[...])),
                              wiou2_ref[...])), wiou3_ref[...]) + biou3_ref[...]
    reg = dot(jax.nn.relu(dot(jax.nn.relu(dot(h, wreg1_ref[...])),
                              wreg2_ref[...])), wreg3_ref[...]) + breg3_ref[...]
    out_ref[...] = jnp.concatenate([cls, iou, reg], axis=1)


def kernel(point_coords, point_features, rois, wg0a, wg0b, wg1a, wg1b,
           wfc1, wfc2, wcls1, wcls2, wcls3, bcls3, wiou1, wiou2, wiou3,
           biou3, wreg1, wreg2, wreg3, breg3):
    f32 = jnp.float32
    x = jnp.concatenate([point_coords, point_features], axis=1)

    gidx = jnp.stack(jnp.meshgrid(jnp.arange(4), jnp.arange(4), jnp.arange(4),
                                  indexing='ij'), axis=-1).reshape(-1, 3)
    gidx = gidx.astype(f32)
    size = rois[:, 3:6]
    local = (gidx[None] + 0.5) / 4 * size[:, None] - size[:, None] / 2
    ang = rois[:, 6]
    c = jnp.cos(ang)
    s = jnp.sin(ang)
    z = jnp.zeros_like(c)
    o = jnp.ones_like(c)
    rot = jnp.stack([c, s, z, -s, c, z, z, z, o], axis=1).reshape(-1, 3, 3)
    gp = jnp.einsum('bpc,bcd->bpd', local, rot) + rois[:, None, 0:3]
    nxyz = gp.reshape(NQ, 3)

    pb, tab, qa = pl.pallas_call(
        _prep_kernel,
        grid=(NP // PBLK,),
        in_specs=[
            pl.BlockSpec((PBLK, 131), lambda i: (i, 0)),
            pl.BlockSpec((PBLK, 3), lambda i: (i, 0)),
            pl.BlockSpec((131, 64), lambda i: (0, 0)),
            pl.BlockSpec((131, 64), lambda i: (0, 0)),
        ],
        out_specs=[
            pl.BlockSpec((PBLK, 8), lambda i: (i, 0)),
            pl.BlockSpec((PBLK, 128), lambda i: (i, 0)),
            pl.BlockSpec((PBLK, 8), lambda i: (i, 0)),
        ],
        out_shape=[
            jax.ShapeDtypeStruct((NP, 8), f32),
            jax.ShapeDtypeStruct((NP, 128), f32),
            jax.ShapeDtypeStruct((NQ, 8), f32),
        ],
    )(x, nxyz, wg0a, wg1a)

    idx48, vmask = pl.pallas_call(
        _select_kernel,
        grid=(NBLK,),
        in_specs=[
            pl.BlockSpec((QBLK, 8), lambda i: (i, 0)),
            pl.BlockSpec((NP, 8), lambda i: (0, 0)),
        ],
        out_specs=[
            pl.BlockSpec((QBLK, K1), lambda i: (i, 0)),
            pl.BlockSpec((QBLK, KT), lambda i: (i, 0)),
        ],
        out_shape=[
            jax.ShapeDtypeStruct((NQ, K1), jnp.int32),
            jax.ShapeDtypeStruct((NQ, KT), f32),
        ],
    )(qa, pb)

    idx_flat = idx48.reshape(TOTAL)
    rows = _sc_gather(tab, idx_flat)
    rows3 = rows.reshape(NQ, K1, 128)

    pooled = pl.pallas_call(
        _mlp_kernel,
        grid=(NBLK,),
        in_specs=[
            pl.BlockSpec((QBLK, 8), lambda i: (i, 0)),
            pl.BlockSpec((QBLK, K1, 128), lambda i: (i, 0, 0)),
            pl.BlockSpec((QBLK, KT), lambda i: (i, 0)),
            pl.BlockSpec((131, 64), lambda i: (0, 0)),
            pl.BlockSpec((131, 64), lambda i: (0, 0)),
            pl.BlockSpec((64, 64), lambda i: (0, 0)),
            pl.BlockSpec((64, 64), lambda i: (0, 0)),
        ],
        out_specs=pl.BlockSpec((QBLK, 128), lambda i: (i, 0)),
        out_shape=jax.ShapeDtypeStruct((NQ, 128), f32),
    )(qa, rows3, vmask, wg0a, wg1a, wg0b, wg1b)

    xfc = pooled.reshape(128, 64 * 128)
    out = pl.pallas_call(
        _heads_kernel,
        out_shape=jax.ShapeDtypeStruct((128, 9), f32),
    )(xfc, wfc1, wfc2, wcls1, wcls2, wcls3, bcls3.reshape(1, 1),
      wiou1, wiou2, wiou3, biou3.reshape(1, 1),
      wreg1, wreg2, wreg3, breg3.reshape(1, 7))
    return out
